# Initial kernel scaffold; baseline (speedup 1.0000x reference)
import jax
import jax.numpy as jnp
from jax.experimental import pallas as pl


def kernel(x, edge_index, edge_weight):
    raise NotImplementedError("write your pallas kernel here")


# speedup vs baseline: 4.0341x; 4.0341x over previous
import jax
import jax.numpy as jnp
from jax import lax
from jax.experimental import pallas as pl
from jax.experimental.pallas import tpu as pltpu
from jax.experimental.pallas import tpu_sc as plsc

_N = 10000
_D = 128
_E = 320000
_NC = 2
_NS = 16
_NW = _NC * _NS
_EPW = _E // _NW
_C = 80
_NCHUNK = _EPW // _C
_NP = 10240
_RPT = _NP // _NS


def _sc_body(x_hbm, src_hbm, dst_hbm, w_hbm, z_hbm, out0, out1,
             src_v, dst_v, w_v, rows_v, acc, sem):
    c = lax.axis_index("c")
    s = lax.axis_index("s")
    wid = s * _NC + c
    row0 = s * _RPT

    pltpu.sync_copy(z_hbm.at[pl.ds(row0, _RPT)], acc.at[pl.ds(row0, _RPT)])
    plsc.subcore_barrier()

    def chunk(g, carry):
        base = wid * _EPW + g * _C
        pltpu.sync_copy(src_hbm.at[pl.ds(base, _C)], src_v)
        pltpu.sync_copy(dst_hbm.at[pl.ds(base, _C)], dst_v)
        pltpu.sync_copy(w_hbm.at[pl.ds(base, _C)], w_v)
        pltpu.async_copy(x_hbm.at[src_v], rows_v, sem).wait()

        def row(i, cc):
            wb = plsc.load_gather(w_v, [jnp.zeros((16,), jnp.int32) + i])
            for j in range(_D // 16):
                sl = pl.ds(j * 16, 16)
                rows_v[i, sl] = rows_v[i, sl] * wb
            return cc

        lax.fori_loop(0, _C, row, 0)
        pltpu.sync_copy(rows_v, acc.at[dst_v], add=True)
        return carry

    lax.fori_loop(0, _NCHUNK, chunk, 0)
    plsc.subcore_barrier()

    @pl.when(c == 0)
    def _():
        pltpu.sync_copy(acc.at[pl.ds(row0, _RPT)], out0.at[pl.ds(row0, _RPT)])

    @pl.when(c == 1)
    def _():
        pltpu.sync_copy(acc.at[pl.ds(row0, _RPT)], out1.at[pl.ds(row0, _RPT)])


def _combine_body(a_ref, b_ref, o_ref):
    t = a_ref[...] + b_ref[...]
    o_ref[...] = 1.0 / (1.0 + jnp.exp(-t))


@jax.jit
def kernel(x, edge_index, edge_weight):
    src = edge_index[0]
    dst = edge_index[1]
    zeros = jnp.zeros((_NP, _D), jnp.float32)
    sc = pl.kernel(
        _sc_body,
        mesh=plsc.VectorSubcoreMesh(core_axis_name="c", subcore_axis_name="s"),
        out_type=[jax.ShapeDtypeStruct((_NP, _D), jnp.float32)] * 2,
        scratch_types=[
            pltpu.VMEM((_C,), jnp.int32),
            pltpu.VMEM((_C,), jnp.int32),
            pltpu.VMEM((_C,), jnp.float32),
            pltpu.VMEM((_C, _D), jnp.float32),
            pltpu.VMEM_SHARED((_NP, _D), jnp.float32),
            pltpu.SemaphoreType.DMA,
        ],
        compiler_params=pltpu.CompilerParams(needs_layout_passes=False),
    )
    o0, o1 = sc(x, src, dst, edge_weight, zeros)

    blk = 1000
    return pl.pallas_call(
        _combine_body,
        out_shape=jax.ShapeDtypeStruct((_N, _D), jnp.float32),
        grid=(_N // blk,),
        in_specs=[pl.BlockSpec((blk, _D), lambda i: (i, 0))] * 2,
        out_specs=pl.BlockSpec((blk, _D), lambda i: (i, 0)),
    )(o0, o1)
